# baseline (speedup 1.0000x reference)
import functools

import jax
import jax.numpy as jnp
from jax import lax
from jax.experimental import pallas as pl
from jax.experimental.pallas import tpu as pltpu
from jax.experimental.pallas import tpu_sc as plsc

_N = 100000
_E = 6400000
_ROWS = 784
_NPAD = _ROWS * 128
_NC = 2
_NS = 16
_NW = _NC * _NS
_EPW = _E // _NW
_K = 2000
_STEPS = _EPW // _K
_SL = _NPAD // _NS
_GAMMA = 0.5

_mesh = plsc.VectorSubcoreMesh(core_axis_name="c", subcore_axis_name="s")


@functools.partial(
    pl.kernel,
    mesh=_mesh,
    compiler_params=pltpu.CompilerParams(
        needs_layout_passes=False, use_tc_tiling_on_sc=False
    ),
    out_type=[
        jax.ShapeDtypeStruct((_NC, _NPAD), jnp.float32),
        jax.ShapeDtypeStruct((_NC, _NPAD), jnp.float32),
    ],
    scratch_types=[
        pltpu.VMEM((_NPAD,), jnp.float32),
        pltpu.VMEM((2, _K), jnp.int32),
        pltpu.VMEM((2, _K), jnp.int32),
        pltpu.VMEM((2, _K), jnp.float32),
        pltpu.VMEM((_K,), jnp.float32),
        pltpu.VMEM_SHARED((_NPAD,), jnp.float32),
        pltpu.VMEM_SHARED((_NPAD,), jnp.float32),
        pltpu.SemaphoreType.DMA,
        pltpu.SemaphoreType.DMA,
        pltpu.SemaphoreType.DMA,
        pltpu.SemaphoreType.DMA,
        pltpu.SemaphoreType.DMA,
        pltpu.SemaphoreType.DMA,
        pltpu.SemaphoreType.DMA,
        pltpu.SemaphoreType.DMA,
    ],
)
def _edge_pass(ei_hbm, mask_hbm, t_hbm, c_hbm,
               mask_v, i0_v, i1_v, val_v, ones_v, t_sh, c_sh,
               sl0_a, sl0_b, sl1_a, sl1_b, st_a, st_b, sc_a, sc_b):
    cid = lax.axis_index("c")
    sid = lax.axis_index("s")
    wid = sid * _NC + cid
    sl0 = (sl0_a, sl0_b)
    sl1 = (sl1_a, sl1_b)
    st = (st_a, st_b)
    sc = (sc_a, sc_b)

    pltpu.sync_copy(mask_hbm, mask_v)

    zeros16 = jnp.zeros((16,), jnp.float32)
    ones16 = jnp.ones((16,), jnp.float32)

    def _fill_z(i, carry):
        o = pl.multiple_of(i * 16, 16)
        val_v[0, pl.ds(o, 16)] = zeros16
        ones_v[pl.ds(o, 16)] = ones16
        return carry

    lax.fori_loop(0, _K // 16, _fill_z, 0)

    off = pl.multiple_of(sid * _SL, 8)
    pos = 0
    while pos < _SL:
        n = min(_K, _SL - pos)
        pltpu.sync_copy(val_v.at[0, pl.ds(0, n)], t_sh.at[pl.ds(off + pos, n)])
        pltpu.sync_copy(val_v.at[0, pl.ds(0, n)], c_sh.at[pl.ds(off + pos, n)])
        pos += n
    plsc.subcore_barrier()

    ebase = wid * _EPW

    def _load(s, p):
        base = pl.multiple_of(ebase + s * _K, 8)
        pltpu.async_copy(ei_hbm.at[pl.ds(base, _K)], i0_v.at[p], sl0[p])
        pltpu.async_copy(ei_hbm.at[pl.ds(base + _E, _K)], i1_v.at[p], sl1[p])

    def _wait_load(s, p):
        base = pl.multiple_of(ebase + s * _K, 8)
        pltpu.make_async_copy(ei_hbm.at[pl.ds(base, _K)], i0_v.at[p], sl0[p]).wait()
        pltpu.make_async_copy(ei_hbm.at[pl.ds(base + _E, _K)], i1_v.at[p], sl1[p]).wait()

    def _gather(p):
        def _g(j, c2):
            base_o = pl.multiple_of(j * 160, 16)
            for u in range(10):
                o = base_o + u * 16
                idx = i1_v[p, pl.ds(o, 16)]
                val_v[p, pl.ds(o, 16)] = plsc.load_gather(mask_v, [idx])
            return c2

        lax.fori_loop(0, _K // 160, _g, 0)
        for u in range(_K // 160 * 10, _K // 16):
            o = u * 16
            idx = i1_v[p, pl.ds(o, 16)]
            val_v[p, pl.ds(o, 16)] = plsc.load_gather(mask_v, [idx])

    def _scatter(p):
        pltpu.async_copy(val_v.at[p], t_sh.at[i0_v.at[p]], st[p], add=True)
        pltpu.async_copy(ones_v, c_sh.at[i0_v.at[p]], sc[p], add=True)

    def _wait_scatter(p):
        pltpu.make_async_copy(val_v.at[p], t_sh.at[i0_v.at[p]], st[p]).wait()
        pltpu.make_async_copy(ones_v, c_sh.at[i0_v.at[p]], sc[p]).wait()

    _load(0, 0)

    def _iter(g, carry):
        s0 = g * 2
        _wait_load(s0, 0)

        @pl.when(g > 0)
        def _():
            _wait_scatter(1)

        _load(s0 + 1, 1)
        _gather(0)
        _scatter(0)
        _wait_load(s0 + 1, 1)
        _wait_scatter(0)

        @pl.when(g + 1 < _STEPS // 2)
        def _():
            _load(s0 + 2, 0)

        _gather(1)
        _scatter(1)
        return carry

    lax.fori_loop(0, _STEPS // 2, _iter, 0)
    _wait_scatter(1)

    plsc.subcore_barrier()
    pltpu.sync_copy(t_sh.at[pl.ds(off, _SL)], t_hbm.at[cid, pl.ds(off, _SL)])
    pltpu.sync_copy(c_sh.at[pl.ds(off, _SL)], c_hbm.at[cid, pl.ds(off, _SL)])


def _fin_body(m_ref, t_ref, c_ref, o_ref):
    m = m_ref[...]
    t = t_ref[...]
    c = c_ref[...]
    ts = t[0] + t[1]
    cs = c[0] + c[1]
    o_ref[...] = (1.0 - _GAMMA) * m + (_GAMMA * 0.5) * (cs * m + ts) / jnp.maximum(cs, 1.0)


_finalize = pl.pallas_call(
    _fin_body,
    out_shape=jax.ShapeDtypeStruct((_ROWS, 128), jnp.float32),
)


def kernel(mask, edge_index, assign_edge):
    del assign_edge
    mask_pad = jnp.pad(mask.reshape(-1), (0, _NPAD - _N))
    t, c = _edge_pass(edge_index.reshape(-1), mask_pad)
    out = _finalize(
        mask_pad.reshape(_ROWS, 128),
        t.reshape(_NC, _ROWS, 128),
        c.reshape(_NC, _ROWS, 128),
    )
    return out.reshape(-1)[:_N].reshape(_N, 1)


# speedup vs baseline: 1.2350x; 1.2350x over previous
import functools

import jax
import jax.numpy as jnp
from jax import lax
from jax.experimental import pallas as pl
from jax.experimental.pallas import tpu as pltpu
from jax.experimental.pallas import tpu_sc as plsc

_N = 100000
_E = 6400000
_ROWS = 784
_NPAD = _ROWS * 128
_NC = 2
_NS = 16
_NW = _NC * _NS
_EPW = _E // _NW
_K = 2000
_STEPS = _EPW // _K
_SL = _NPAD // _NS
_GAMMA = 0.5

_mesh = plsc.VectorSubcoreMesh(core_axis_name="c", subcore_axis_name="s")


@functools.partial(
    pl.kernel,
    mesh=_mesh,
    compiler_params=pltpu.CompilerParams(
        needs_layout_passes=False, use_tc_tiling_on_sc=False
    ),
    out_type=[
        jax.ShapeDtypeStruct((_NC, _NPAD), jnp.float32),
        jax.ShapeDtypeStruct((_NC, _NPAD), jnp.float32),
    ],
    scratch_types=[
        pltpu.VMEM((_NPAD,), jnp.float32),
        pltpu.VMEM((2, _K), jnp.int32),
        pltpu.VMEM((2, _K), jnp.int32),
        pltpu.VMEM((2, _K), jnp.float32),
        pltpu.VMEM((_K,), jnp.float32),
        pltpu.VMEM_SHARED((_NPAD,), jnp.float32),
        pltpu.VMEM_SHARED((_NPAD,), jnp.float32),
        pltpu.SemaphoreType.DMA,
        pltpu.SemaphoreType.DMA,
        pltpu.SemaphoreType.DMA,
        pltpu.SemaphoreType.DMA,
        pltpu.SemaphoreType.DMA,
        pltpu.SemaphoreType.DMA,
        pltpu.SemaphoreType.DMA,
        pltpu.SemaphoreType.DMA,
        pltpu.SemaphoreType.DMA,
        pltpu.SemaphoreType.DMA,
        pltpu.SemaphoreType.DMA,
        pltpu.SemaphoreType.DMA,
    ],
)
def _edge_pass(ei_hbm, mask_hbm, t_hbm, c_hbm,
               mask_v, i0_v, i1_v, val_v, ones_v, t_sh, c_sh,
               sl0_a, sl0_b, sl1_a, sl1_b,
               st_a0, st_a1, st_b0, st_b1, sc_a0, sc_a1, sc_b0, sc_b1):
    cid = lax.axis_index("c")
    sid = lax.axis_index("s")
    wid = sid * _NC + cid
    sl0 = (sl0_a, sl0_b)
    sl1 = (sl1_a, sl1_b)
    st = ((st_a0, st_a1), (st_b0, st_b1))
    sc = ((sc_a0, sc_a1), (sc_b0, sc_b1))
    _H = _K // 2

    pltpu.sync_copy(mask_hbm, mask_v)

    zeros16 = jnp.zeros((16,), jnp.float32)
    ones16 = jnp.ones((16,), jnp.float32)

    def _fill_z(i, carry):
        o = pl.multiple_of(i * 16, 16)
        val_v[0, pl.ds(o, 16)] = zeros16
        ones_v[pl.ds(o, 16)] = ones16
        return carry

    lax.fori_loop(0, _K // 16, _fill_z, 0)

    off = pl.multiple_of(sid * _SL, 8)
    pos = 0
    while pos < _SL:
        n = min(_K, _SL - pos)
        pltpu.sync_copy(val_v.at[0, pl.ds(0, n)], t_sh.at[pl.ds(off + pos, n)])
        pltpu.sync_copy(val_v.at[0, pl.ds(0, n)], c_sh.at[pl.ds(off + pos, n)])
        pos += n
    plsc.subcore_barrier()

    ebase = wid * _EPW

    def _load(s, p):
        base = pl.multiple_of(ebase + s * _K, 8)
        pltpu.async_copy(ei_hbm.at[pl.ds(base, _K)], i0_v.at[p], sl0[p])
        pltpu.async_copy(ei_hbm.at[pl.ds(base + _E, _K)], i1_v.at[p], sl1[p])

    def _wait_load(s, p):
        base = pl.multiple_of(ebase + s * _K, 8)
        pltpu.make_async_copy(ei_hbm.at[pl.ds(base, _K)], i0_v.at[p], sl0[p]).wait()
        pltpu.make_async_copy(ei_hbm.at[pl.ds(base + _E, _K)], i1_v.at[p], sl1[p]).wait()

    def _gather(p):
        def _g(j, c2):
            base_o = pl.multiple_of(j * 160, 16)
            for u in range(10):
                o = base_o + u * 16
                idx = i1_v[p, pl.ds(o, 16)]
                val_v[p, pl.ds(o, 16)] = plsc.load_gather(mask_v, [idx])
            return c2

        lax.fori_loop(0, _K // 160, _g, 0)
        for u in range(_K // 160 * 10, _K // 16):
            o = u * 16
            idx = i1_v[p, pl.ds(o, 16)]
            val_v[p, pl.ds(o, 16)] = plsc.load_gather(mask_v, [idx])

    def _scatter(p):
        for h in range(2):
            idx = i0_v.at[p, pl.ds(h * _H, _H)]
            pltpu.async_copy(val_v.at[p, pl.ds(h * _H, _H)], t_sh.at[idx],
                             st[p][h], add=True)
            pltpu.async_copy(ones_v.at[pl.ds(h * _H, _H)], c_sh.at[idx],
                             sc[p][h], add=True)

    def _wait_scatter(p):
        for h in range(2):
            idx = i0_v.at[p, pl.ds(h * _H, _H)]
            pltpu.make_async_copy(val_v.at[p, pl.ds(h * _H, _H)], t_sh.at[idx],
                                  st[p][h]).wait()
            pltpu.make_async_copy(ones_v.at[pl.ds(h * _H, _H)], c_sh.at[idx],
                                  sc[p][h]).wait()

    _load(0, 0)

    def _iter(g, carry):
        s0 = g * 2
        _wait_load(s0, 0)
        _gather(0)

        @pl.when(g > 0)
        def _():
            _wait_scatter(1)

        _load(s0 + 1, 1)
        _scatter(0)
        _wait_load(s0 + 1, 1)
        _gather(1)
        _wait_scatter(0)

        @pl.when(g + 1 < _STEPS // 2)
        def _():
            _load(s0 + 2, 0)

        _scatter(1)
        return carry

    lax.fori_loop(0, _STEPS // 2, _iter, 0)
    _wait_scatter(1)

    plsc.subcore_barrier()
    pltpu.sync_copy(t_sh.at[pl.ds(off, _SL)], t_hbm.at[cid, pl.ds(off, _SL)])
    pltpu.sync_copy(c_sh.at[pl.ds(off, _SL)], c_hbm.at[cid, pl.ds(off, _SL)])


def _fin_body(m_ref, t_ref, c_ref, o_ref):
    m = m_ref[...]
    t = t_ref[...]
    c = c_ref[...]
    ts = t[0] + t[1]
    cs = c[0] + c[1]
    o_ref[...] = (1.0 - _GAMMA) * m + (_GAMMA * 0.5) * (cs * m + ts) / jnp.maximum(cs, 1.0)


_finalize = pl.pallas_call(
    _fin_body,
    out_shape=jax.ShapeDtypeStruct((_ROWS, 128), jnp.float32),
)


def kernel(mask, edge_index, assign_edge):
    del assign_edge
    mask_pad = jnp.pad(mask.reshape(-1), (0, _NPAD - _N))
    t, c = _edge_pass(edge_index.reshape(-1), mask_pad)
    out = _finalize(
        mask_pad.reshape(_ROWS, 128),
        t.reshape(_NC, _ROWS, 128),
        c.reshape(_NC, _ROWS, 128),
    )
    return out.reshape(-1)[:_N].reshape(_N, 1)


# speedup vs baseline: 1.2410x; 1.0049x over previous
import functools

import jax
import jax.numpy as jnp
from jax import lax
from jax.experimental import pallas as pl
from jax.experimental.pallas import tpu as pltpu
from jax.experimental.pallas import tpu_sc as plsc

_N = 100000
_E = 6400000
_ROWS = 784
_NPAD = _ROWS * 128
_NC = 2
_NS = 16
_NW = _NC * _NS
_EPW = _E // _NW
_K = 1600
_STEPS = _EPW // _K
_NBUF = 3
_SL = _NPAD // _NS
_GAMMA = 0.5

_mesh = plsc.VectorSubcoreMesh(core_axis_name="c", subcore_axis_name="s")


@functools.partial(
    pl.kernel,
    mesh=_mesh,
    compiler_params=pltpu.CompilerParams(
        needs_layout_passes=False, use_tc_tiling_on_sc=False
    ),
    out_type=[
        jax.ShapeDtypeStruct((_NC, _NPAD), jnp.float32),
        jax.ShapeDtypeStruct((_NC, _NPAD), jnp.float32),
    ],
    scratch_types=[
        pltpu.VMEM((_NPAD,), jnp.float32),
        pltpu.VMEM((_NBUF, _K), jnp.int32),
        pltpu.VMEM((_NBUF, _K), jnp.int32),
        pltpu.VMEM((_NBUF, _K), jnp.float32),
        pltpu.VMEM((_K,), jnp.float32),
        pltpu.VMEM_SHARED((_NPAD,), jnp.float32),
        pltpu.VMEM_SHARED((_NPAD,), jnp.float32),
        pltpu.SemaphoreType.DMA,
        pltpu.SemaphoreType.DMA,
        pltpu.SemaphoreType.DMA,
        pltpu.SemaphoreType.DMA,
        pltpu.SemaphoreType.DMA,
        pltpu.SemaphoreType.DMA,
        pltpu.SemaphoreType.DMA,
        pltpu.SemaphoreType.DMA,
        pltpu.SemaphoreType.DMA,
        pltpu.SemaphoreType.DMA,
        pltpu.SemaphoreType.DMA,
        pltpu.SemaphoreType.DMA,
    ],
)
def _edge_pass(ei_hbm, mask_hbm, t_hbm, c_hbm,
               mask_v, i0_v, i1_v, val_v, ones_v, t_sh, c_sh,
               sl0_a, sl0_b, sl0_c, sl1_a, sl1_b, sl1_c,
               st_a, st_b, st_c, sc_a, sc_b, sc_c):
    cid = lax.axis_index("c")
    sid = lax.axis_index("s")
    wid = sid * _NC + cid
    sl0 = (sl0_a, sl0_b, sl0_c)
    sl1 = (sl1_a, sl1_b, sl1_c)
    st = (st_a, st_b, st_c)
    sc = (sc_a, sc_b, sc_c)

    pltpu.sync_copy(mask_hbm, mask_v)

    zeros16 = jnp.zeros((16,), jnp.float32)
    ones16 = jnp.ones((16,), jnp.float32)

    def _fill_z(i, carry):
        o = pl.multiple_of(i * 16, 16)
        val_v[0, pl.ds(o, 16)] = zeros16
        ones_v[pl.ds(o, 16)] = ones16
        return carry

    lax.fori_loop(0, _K // 16, _fill_z, 0)

    off = pl.multiple_of(sid * _SL, 8)
    pos = 0
    while pos < _SL:
        n = min(_K, _SL - pos)
        pltpu.sync_copy(val_v.at[0, pl.ds(0, n)], t_sh.at[pl.ds(off + pos, n)])
        pltpu.sync_copy(val_v.at[0, pl.ds(0, n)], c_sh.at[pl.ds(off + pos, n)])
        pos += n
    plsc.subcore_barrier()

    ebase = wid * _EPW

    def _load(s, p):
        base = pl.multiple_of(ebase + s * _K, 8)
        pltpu.async_copy(ei_hbm.at[pl.ds(base, _K)], i0_v.at[p], sl0[p])
        pltpu.async_copy(ei_hbm.at[pl.ds(base + _E, _K)], i1_v.at[p], sl1[p])

    def _wait_load(s, p):
        base = pl.multiple_of(ebase + s * _K, 8)
        pltpu.make_async_copy(ei_hbm.at[pl.ds(base, _K)], i0_v.at[p], sl0[p]).wait()
        pltpu.make_async_copy(ei_hbm.at[pl.ds(base + _E, _K)], i1_v.at[p], sl1[p]).wait()

    def _gather(p):
        def _g(j, c2):
            base_o = pl.multiple_of(j * 160, 16)
            for u in range(10):
                o = base_o + u * 16
                idx = i1_v[p, pl.ds(o, 16)]
                val_v[p, pl.ds(o, 16)] = plsc.load_gather(mask_v, [idx])
            return c2

        lax.fori_loop(0, _K // 160, _g, 0)

    def _scatter(p):
        pltpu.async_copy(val_v.at[p], t_sh.at[i0_v.at[p]], st[p], add=True)
        pltpu.async_copy(ones_v, c_sh.at[i0_v.at[p]], sc[p], add=True)

    def _wait_scatter(p):
        pltpu.make_async_copy(val_v.at[p], t_sh.at[i0_v.at[p]], st[p]).wait()
        pltpu.make_async_copy(ones_v, c_sh.at[i0_v.at[p]], sc[p]).wait()

    _load(0, 0)
    _load(1, 1)

    def _iter(g, carry):
        for ph in range(_NBUF):
            s = g * _NBUF + ph
            _wait_load(s, ph)
            _gather(ph)
            pprev = (ph + _NBUF - 1) % _NBUF

            if ph == 0:
                @pl.when(g > 0)
                def _():
                    _wait_scatter(pprev)
            else:
                _wait_scatter(pprev)

            _load(s + 2, (ph + 2) % _NBUF)
            _scatter(ph)
        return carry

    lax.fori_loop(0, _STEPS // _NBUF, _iter, 0)

    for s, ph in ((_STEPS - 2, 0), (_STEPS - 1, 1)):
        _wait_load(s, ph)
        _gather(ph)
        _wait_scatter((ph + _NBUF - 1) % _NBUF)
        _scatter(ph)
    _wait_scatter(1)

    plsc.subcore_barrier()
    pltpu.sync_copy(t_sh.at[pl.ds(off, _SL)], t_hbm.at[cid, pl.ds(off, _SL)])
    pltpu.sync_copy(c_sh.at[pl.ds(off, _SL)], c_hbm.at[cid, pl.ds(off, _SL)])


def _fin_body(m_ref, t_ref, c_ref, o_ref):
    m = m_ref[...]
    t = t_ref[...]
    c = c_ref[...]
    ts = t[0] + t[1]
    cs = c[0] + c[1]
    o_ref[...] = (1.0 - _GAMMA) * m + (_GAMMA * 0.5) * (cs * m + ts) / jnp.maximum(cs, 1.0)


_finalize = pl.pallas_call(
    _fin_body,
    out_shape=jax.ShapeDtypeStruct((_ROWS, 128), jnp.float32),
)


def kernel(mask, edge_index, assign_edge):
    del assign_edge
    mask_pad = jnp.pad(mask.reshape(-1), (0, _NPAD - _N))
    t, c = _edge_pass(edge_index.reshape(-1), mask_pad)
    out = _finalize(
        mask_pad.reshape(_ROWS, 128),
        t.reshape(_NC, _ROWS, 128),
        c.reshape(_NC, _ROWS, 128),
    )
    return out.reshape(-1)[:_N].reshape(_N, 1)
